# Initial kernel scaffold; baseline (speedup 1.0000x reference)
import jax
import jax.numpy as jnp
from jax.experimental import pallas as pl


def kernel(nodes, edge_index, edge_attr, action_mask, W1, a_src1, a_dst1, We1, a_e1, b1, W2, a_src2, a_dst2, We2, a_e2, b2, Wl1, bl1, Wl2, bl2, Wc1, bc1, Wc2, bc2):
    raise NotImplementedError("write your pallas kernel here")


# speedup vs baseline: 34.9467x; 34.9467x over previous
import functools

import jax
import jax.numpy as jnp
from jax import lax
from jax.experimental import pallas as pl
from jax.experimental.pallas import tpu as pltpu
from jax.experimental.pallas import tpu_sc as plsc

NODE_NUM = 10000
REG_NUM = 64
N_EDGES = 320000
D_EDGE = 4
HEADS = 4
OUT_CH = 8
HID = HEADS * OUT_CH

NC = 2
NS = 16
NW = NC * NS
CHUNK = 128
CH_PER_TILE = 79
EPT = CH_PER_TILE * CHUNK
NE_PAD = NW * EPT
ACC_ROWS = 10240
ZROWS = ACC_ROWS // NS

def _sc_mesh():
    return plsc.VectorSubcoreMesh(core_axis_name="c", subcore_axis_name="s",
                                  num_cores=NC, num_subcores=NS)


def _sc1_body(src_hbm, dst_hbm, ae_hbm, nodes_hbm, cons_hbm, z8_hbm,
              out_hbm, nodes_v, cons_v, src_b, dst_b, ae_b, pq_b, pq_sh):
    cid = lax.axis_index("c")
    sid = lax.axis_index("s")
    wid = sid * NC + cid
    pltpu.sync_copy(nodes_hbm, nodes_v)
    pltpu.sync_copy(cons_hbm, cons_v)
    pltpu.sync_copy(z8_hbm, pq_sh.at[pl.ds(sid * ZROWS, ZROWS)])
    plsc.subcore_barrier()

    i16 = lax.iota(jnp.int32, 16)
    asv = [cons_v[pl.ds(h * 16, 16)] for h in range(HEADS)]
    adv = [cons_v[pl.ds((4 + h) * 16, 16)] for h in range(HEADS)]
    mv = [cons_v[pl.ds((8 + h) * 16, 16)] for h in range(HEADS)]

    def chunk_body(c, _):
        base = wid * EPT + c * CHUNK
        pltpu.sync_copy(src_hbm.at[pl.ds(base, CHUNK)], src_b)
        pltpu.sync_copy(dst_hbm.at[pl.ds(base, CHUNK)], dst_b)
        pltpu.sync_copy(ae_hbm.at[pl.ds(base, CHUNK)], ae_b)
        for o in range(0, CHUNK, 16):
            srcv = src_b[pl.ds(o, 16)]
            dstv = dst_b[pl.ds(o, 16)]
            rows = i16 + o
            nsrc = plsc.load_gather(nodes_v, [srcv])
            ndst = plsc.load_gather(nodes_v, [dstv])
            for h in range(HEADS):
                hc = jnp.full((16,), h, jnp.int32)
                aev = plsc.load_gather(ae_b, [rows, hc])
                al = nsrc * asv[h] + ndst * adv[h] + aev
                al = jnp.maximum(al, al * 0.2)
                p = jnp.exp(al - mv[h])
                plsc.store_scatter(pq_b, [rows, hc], p)
                plsc.store_scatter(pq_b, [rows, hc + 4], p * nsrc)
        pltpu.sync_copy(pq_b, pq_sh.at[dst_b], add=True)
        return _

    lax.fori_loop(0, CH_PER_TILE, chunk_body, None)
    plsc.subcore_barrier()
    pltpu.sync_copy(pq_sh.at[pl.ds(sid * ZROWS, ZROWS)],
                    out_hbm.at[cid, pl.ds(sid * ZROWS, ZROWS)])


@functools.lru_cache(maxsize=1)
def _make_sc1():
  @functools.partial(
    pl.kernel,
    out_type=jax.ShapeDtypeStruct((NC, ACC_ROWS, 2 * HEADS), jnp.float32),
    mesh=_sc_mesh(),
    compiler_params=pltpu.CompilerParams(needs_layout_passes=False,
                                        use_tc_tiling_on_sc=False),
    scratch_types=[
        pltpu.VMEM((NODE_NUM,), jnp.float32),
        pltpu.VMEM((12 * 16,), jnp.float32),
        pltpu.VMEM((CHUNK,), jnp.int32),
        pltpu.VMEM((CHUNK,), jnp.int32),
        pltpu.VMEM((CHUNK, HEADS), jnp.float32),
        pltpu.VMEM((CHUNK, 2 * HEADS), jnp.float32),
        pltpu.VMEM_SHARED((ACC_ROWS, 2 * HEADS), jnp.float32),
    ],
)
  def _sc_layer1(src_hbm, dst_hbm, ae_hbm, nodes_hbm, cons_hbm, z8_hbm,
                 out_hbm, *scratch):
      _sc1_body(src_hbm, dst_hbm, ae_hbm, nodes_hbm, cons_hbm, z8_hbm,
                out_hbm, *scratch)
  return _sc_layer1


def _sc2_body(src_hbm, dst_hbm, ae_hbm, h2_hbm, as_hbm, ad_hbm, m_hbm,
              z36_hbm, pm_out,
              as_v, ad_v, m_v, src_b, dst_b, ae_b, h_b, pm_b, pm_sh, sem):
    cid = lax.axis_index("c")
    sid = lax.axis_index("s")
    wid = sid * NC + cid
    pltpu.sync_copy(as_hbm, as_v)
    pltpu.sync_copy(ad_hbm, ad_v)
    pltpu.sync_copy(m_hbm, m_v)
    pltpu.sync_copy(z36_hbm, pm_sh.at[pl.ds(sid * ZROWS, ZROWS)])
    plsc.subcore_barrier()

    i16 = lax.iota(jnp.int32, 16)
    mv = [m_v[pl.ds(h * 16, 16)] for h in range(HEADS)]
    zv = jnp.zeros((16,), jnp.float32)
    for o in range(0, CHUNK, 16):
        for pc in range(36, 40):
            plsc.store_scatter(pm_b, [i16 + o, jnp.full((16,), pc, jnp.int32)], zv)

    def chunk_body(c, _):
        base = wid * EPT + c * CHUNK
        pltpu.sync_copy(src_hbm.at[pl.ds(base, CHUNK)], src_b)
        pltpu.sync_copy(dst_hbm.at[pl.ds(base, CHUNK)], dst_b)
        pltpu.sync_copy(ae_hbm.at[pl.ds(base, CHUNK)], ae_b)
        pltpu.async_copy(h2_hbm.at[src_b], h_b, sem).wait()
        for o in range(0, CHUNK, 16):
            srcv = src_b[pl.ds(o, 16)]
            dstv = dst_b[pl.ds(o, 16)]
            srcv4 = srcv * 4
            dstv4 = dstv * 4
            rows = i16 + o
            for h in range(HEADS):
                hc = jnp.full((16,), h, jnp.int32)
                aev = plsc.load_gather(ae_b, [rows, hc])
                av = plsc.load_gather(as_v, [srcv4 + h])
                dv = plsc.load_gather(ad_v, [dstv4 + h])
                al = av + dv + aev
                al = jnp.maximum(al, al * 0.2)
                p = jnp.exp(al - mv[h])
                plsc.store_scatter(pm_b, [rows, hc], p)
                for cc in range(OUT_CH):
                    col = jnp.full((16,), 4 + h * OUT_CH + cc, jnp.int32)
                    hval = plsc.load_gather(h_b, [rows, jnp.full((16,), h * OUT_CH + cc, jnp.int32)])
                    plsc.store_scatter(pm_b, [rows, col], p * hval)
        pltpu.sync_copy(pm_b, pm_sh.at[dst_b], add=True)
        return _

    lax.fori_loop(0, CH_PER_TILE, chunk_body, None)
    plsc.subcore_barrier()
    pltpu.sync_copy(pm_sh.at[pl.ds(sid * ZROWS, ZROWS)],
                    pm_out.at[cid, pl.ds(sid * ZROWS, ZROWS)])


@functools.lru_cache(maxsize=1)
def _make_sc2():
  @functools.partial(
    pl.kernel,
    out_type=jax.ShapeDtypeStruct((NC, ACC_ROWS, 40), jnp.float32),
    mesh=_sc_mesh(),
    compiler_params=pltpu.CompilerParams(needs_layout_passes=False,
                                        use_tc_tiling_on_sc=False),
    scratch_types=[
        pltpu.VMEM((ACC_ROWS * HEADS,), jnp.float32),
        pltpu.VMEM((ACC_ROWS * HEADS,), jnp.float32),
        pltpu.VMEM((4 * 16,), jnp.float32),
        pltpu.VMEM((CHUNK,), jnp.int32),
        pltpu.VMEM((CHUNK,), jnp.int32),
        pltpu.VMEM((CHUNK, HEADS), jnp.float32),
        pltpu.VMEM((CHUNK, HID), jnp.float32),
        pltpu.VMEM((CHUNK, 40), jnp.float32),
        pltpu.VMEM_SHARED((ACC_ROWS, 40), jnp.float32),
        pltpu.SemaphoreType.DMA,
    ],
)
  def _sc_layer2(src_hbm, dst_hbm, ae_hbm, h2_hbm, as_hbm, ad_hbm, m_hbm,
                 z36_hbm, pm_out, *scratch):
      _sc2_body(src_hbm, dst_hbm, ae_hbm, h2_hbm, as_hbm, ad_hbm, m_hbm,
                z36_hbm, pm_out, *scratch)
  return _sc_layer2


_E_BLK = 3200
_E_NBLK = N_EDGES // _E_BLK


def _prep0_body(ea_ref, ae1m_ref, ae2m_ref, nodes_ref,
                ae1_ref, ae2_ref, mx1_ref, mx2_ref, nmx_ref, nmn_ref,
                acc1, acc2):
    i = pl.program_id(0)
    ea = ea_ref[...]
    a1 = jnp.dot(ea, ae1m_ref[...], preferred_element_type=jnp.float32)
    a2 = jnp.dot(ea, ae2m_ref[...], preferred_element_type=jnp.float32)
    ae1_ref[...] = a1
    ae2_ref[...] = a2

    @pl.when(i == 0)
    def _():
        acc1[...] = jnp.full_like(acc1, -1e30)
        acc2[...] = jnp.full_like(acc2, -1e30)
        nodes = nodes_ref[...]
        nmx_ref[...] = jnp.max(nodes, axis=0, keepdims=True)
        nmn_ref[...] = jnp.min(nodes, axis=0, keepdims=True)

    acc1[...] = jnp.maximum(acc1[...], jnp.max(a1, axis=0, keepdims=True))
    acc2[...] = jnp.maximum(acc2[...], jnp.max(a2, axis=0, keepdims=True))

    @pl.when(i == _E_NBLK - 1)
    def _():
        mx1_ref[...] = acc1[...]
        mx2_ref[...] = acc2[...]


@jax.jit
def _prep0(ea, ae1m, ae2m, nodes):
    return pl.pallas_call(
        _prep0_body,
        grid=(_E_NBLK,),
        in_specs=[
            pl.BlockSpec((_E_BLK, D_EDGE), lambda i: (i, 0)),
            pl.BlockSpec((D_EDGE, HEADS), lambda i: (0, 0)),
            pl.BlockSpec((D_EDGE, HEADS), lambda i: (0, 0)),
            pl.BlockSpec((NODE_NUM, 1), lambda i: (0, 0)),
        ],
        out_specs=[
            pl.BlockSpec((_E_BLK, HEADS), lambda i: (i, 0)),
            pl.BlockSpec((_E_BLK, HEADS), lambda i: (i, 0)),
            pl.BlockSpec((1, HEADS), lambda i: (0, 0)),
            pl.BlockSpec((1, HEADS), lambda i: (0, 0)),
            pl.BlockSpec((1, 1), lambda i: (0, 0)),
            pl.BlockSpec((1, 1), lambda i: (0, 0)),
        ],
        out_shape=[
            jax.ShapeDtypeStruct((N_EDGES, HEADS), jnp.float32),
            jax.ShapeDtypeStruct((N_EDGES, HEADS), jnp.float32),
            jax.ShapeDtypeStruct((1, HEADS), jnp.float32),
            jax.ShapeDtypeStruct((1, HEADS), jnp.float32),
            jax.ShapeDtypeStruct((1, 1), jnp.float32),
            jax.ShapeDtypeStruct((1, 1), jnp.float32),
        ],
        scratch_shapes=[
            pltpu.VMEM((1, HEADS), jnp.float32),
            pltpu.VMEM((1, HEADS), jnp.float32),
        ],
    )(ea, ae1m, ae2m, nodes)


_N_BLK = 1280
_N_NBLK = ACC_ROWS // _N_BLK


def _prep2_body(pq_ref, w1_ref, b1_ref, dm1_ref, e4_ref, w2_ref,
                as2m_ref, ad2m_ref,
                h2_ref, as2_ref, ad2_ref, mxs_ref, mxd_ref, accs, accd):
    i = pl.program_id(0)
    pq = pq_ref[0] + pq_ref[1]
    s1 = pq[:, :HEADS]
    q = pq[:, HEADS:]
    e4 = e4_ref[...]
    s1e = jnp.dot(s1, e4, preferred_element_type=jnp.float32)
    qe = jnp.dot(q, e4, preferred_element_type=jnp.float32)
    x1 = w1_ref[...] * qe / (s1e + 1e-16) + b1_ref[...]
    x1 = jnp.where(x1 > 0, x1, jnp.exp(x1) - 1.0)
    x1 = x1 * dm1_ref[...]
    h2 = jnp.dot(x1, w2_ref[...], preferred_element_type=jnp.float32)
    h2_ref[...] = h2
    as2 = jnp.dot(h2, as2m_ref[...], preferred_element_type=jnp.float32)
    ad2 = jnp.dot(h2, ad2m_ref[...], preferred_element_type=jnp.float32)
    as2_ref[...] = as2
    ad2_ref[...] = ad2

    @pl.when(i == 0)
    def _():
        accs[...] = jnp.full_like(accs, -1e30)
        accd[...] = jnp.full_like(accd, -1e30)

    accs[...] = jnp.maximum(accs[...], jnp.max(as2, axis=0, keepdims=True))
    accd[...] = jnp.maximum(accd[...], jnp.max(ad2, axis=0, keepdims=True))

    @pl.when(i == _N_NBLK - 1)
    def _():
        mxs_ref[...] = accs[...]
        mxd_ref[...] = accd[...]


@jax.jit
def _prep2(pq_part, w1, b1, dm1, e4, w2, as2m, ad2m):
    return pl.pallas_call(
        _prep2_body,
        grid=(_N_NBLK,),
        in_specs=[
            pl.BlockSpec((NC, _N_BLK, 2 * HEADS), lambda i: (0, i, 0)),
            pl.BlockSpec((1, HID), lambda i: (0, 0)),
            pl.BlockSpec((1, HID), lambda i: (0, 0)),
            pl.BlockSpec((_N_BLK, HID), lambda i: (i, 0)),
            pl.BlockSpec((HEADS, HID), lambda i: (0, 0)),
            pl.BlockSpec((HID, HID), lambda i: (0, 0)),
            pl.BlockSpec((HID, HEADS), lambda i: (0, 0)),
            pl.BlockSpec((HID, HEADS), lambda i: (0, 0)),
        ],
        out_specs=[
            pl.BlockSpec((_N_BLK, HID), lambda i: (i, 0)),
            pl.BlockSpec((_N_BLK, HEADS), lambda i: (i, 0)),
            pl.BlockSpec((_N_BLK, HEADS), lambda i: (i, 0)),
            pl.BlockSpec((1, HEADS), lambda i: (0, 0)),
            pl.BlockSpec((1, HEADS), lambda i: (0, 0)),
        ],
        out_shape=[
            jax.ShapeDtypeStruct((ACC_ROWS, HID), jnp.float32),
            jax.ShapeDtypeStruct((ACC_ROWS, HEADS), jnp.float32),
            jax.ShapeDtypeStruct((ACC_ROWS, HEADS), jnp.float32),
            jax.ShapeDtypeStruct((1, HEADS), jnp.float32),
            jax.ShapeDtypeStruct((1, HEADS), jnp.float32),
        ],
        scratch_shapes=[
            pltpu.VMEM((1, HEADS), jnp.float32),
            pltpu.VMEM((1, HEADS), jnp.float32),
        ],
    )(pq_part, w1, b1, dm1, e4, w2, as2m, ad2m)


def _prep3_body(pm_ref, b2_ref, dm2_ref, e4_ref, x2_ref):
    pm = pm_ref[0] + pm_ref[1]
    s2 = pm[:, :HEADS]
    acc = pm[:, HEADS:HEADS + HID]
    s2e = jnp.dot(s2, e4_ref[...], preferred_element_type=jnp.float32)
    x2 = acc / (s2e + 1e-16) + b2_ref[...]
    x2 = jnp.where(x2 > 0, x2, jnp.exp(x2) - 1.0)
    x2_ref[...] = x2 * dm2_ref[...]


@jax.jit
def _prep3(pm_part, b2, dm2, e4):
    return pl.pallas_call(
        _prep3_body,
        grid=(_N_NBLK,),
        in_specs=[
            pl.BlockSpec((NC, _N_BLK, 40), lambda i: (0, i, 0)),
            pl.BlockSpec((1, HID), lambda i: (0, 0)),
            pl.BlockSpec((_N_BLK, HID), lambda i: (i, 0)),
            pl.BlockSpec((HEADS, HID), lambda i: (0, 0)),
        ],
        out_specs=pl.BlockSpec((_N_BLK, HID), lambda i: (i, 0)),
        out_shape=jax.ShapeDtypeStruct((ACC_ROWS, HID), jnp.float32),
    )(pm_part, b2, dm2, e4)


K_TOTAL = NODE_NUM * HID
K_BLK = 6400
N_BLKS = K_TOTAL // K_BLK


def _head_body(x_ref, wl1_ref, wc1_ref, bl1_ref, wl2_ref, bl2_ref,
               bc1_ref, wc2_ref, bc2_ref, mask_ref,
               probs_ref, value_ref, acc1_ref, acc2_ref):
    i = pl.program_id(0)

    @pl.when(i == 0)
    def _():
        acc1_ref[...] = jnp.zeros_like(acc1_ref)
        acc2_ref[...] = jnp.zeros_like(acc2_ref)

    x = x_ref[...]
    acc1_ref[...] += jnp.dot(x, wl1_ref[...], preferred_element_type=jnp.float32)
    acc2_ref[...] += jnp.dot(x, wc1_ref[...], preferred_element_type=jnp.float32)

    @pl.when(i == N_BLKS - 1)
    def _():
        y1 = acc1_ref[...] + bl1_ref[...]
        p = jnp.dot(y1, wl2_ref[...], preferred_element_type=jnp.float32) + bl2_ref[...]
        p = jnp.tanh(p)
        p = jnp.where(mask_ref[...] > 0, p, -999999.0)
        m = jnp.max(p, axis=-1, keepdims=True)
        e = jnp.exp(p - m)
        probs_ref[...] = e / jnp.sum(e, axis=-1, keepdims=True)
        y2 = acc2_ref[...] + bc1_ref[...]
        value_ref[...] = (jnp.dot(y2, wc2_ref[...], preferred_element_type=jnp.float32)
                          + bc2_ref[...])


@jax.jit
def _dense_head(xf, Wl1, bl1, Wl2, bl2, Wc1, bc1, Wc2, bc2, maskf):
    return pl.pallas_call(
        _head_body,
        grid=(N_BLKS,),
        in_specs=[
            pl.BlockSpec((1, K_BLK), lambda i: (0, i)),
            pl.BlockSpec((K_BLK, 2 * REG_NUM), lambda i: (i, 0)),
            pl.BlockSpec((K_BLK, REG_NUM), lambda i: (i, 0)),
            pl.BlockSpec((1, 2 * REG_NUM), lambda i: (0, 0)),
            pl.BlockSpec((2 * REG_NUM, REG_NUM), lambda i: (0, 0)),
            pl.BlockSpec((1, REG_NUM), lambda i: (0, 0)),
            pl.BlockSpec((1, REG_NUM), lambda i: (0, 0)),
            pl.BlockSpec((REG_NUM, 1), lambda i: (0, 0)),
            pl.BlockSpec((1, 1), lambda i: (0, 0)),
            pl.BlockSpec((1, REG_NUM), lambda i: (0, 0)),
        ],
        out_specs=[
            pl.BlockSpec((1, REG_NUM), lambda i: (0, 0)),
            pl.BlockSpec((1, 1), lambda i: (0, 0)),
        ],
        out_shape=[
            jax.ShapeDtypeStruct((1, REG_NUM), jnp.float32),
            jax.ShapeDtypeStruct((1, 1), jnp.float32),
        ],
        scratch_shapes=[
            pltpu.VMEM((1, 2 * REG_NUM), jnp.float32),
            pltpu.VMEM((1, REG_NUM), jnp.float32),
        ],
    )(xf, Wl1, Wc1, bl1, Wl2, bl2, bc1, Wc2, bc2, maskf)


def _expander():
    e4 = jnp.zeros((HEADS, HID), jnp.float32)
    idx = jnp.arange(HID)
    return e4.at[idx // OUT_CH, idx].set(1.0)


def _head_embed(a):
    m = jnp.zeros((HID, HEADS), jnp.float32)
    idx = jnp.arange(HID)
    return m.at[idx, idx // OUT_CH].set(a.reshape(-1)[idx])


def _splat16(rows):
    return jnp.concatenate([jnp.repeat(r, 16) for r in rows])


def kernel(nodes, edge_index, edge_attr, action_mask, W1, a_src1, a_dst1, We1,
           a_e1, b1, W2, a_src2, a_dst2, We2, a_e2, b2, Wl1, bl1, Wl2, bl2,
           Wc1, bc1, Wc2, bc2):
    f32 = jnp.float32
    src = edge_index[0].astype(jnp.int32)
    dst = edge_index[1].astype(jnp.int32)
    npad = NE_PAD - N_EDGES
    srcp = jnp.concatenate([src, jnp.zeros((npad,), jnp.int32)])
    dstp = jnp.concatenate([dst, jnp.zeros((npad,), jnp.int32)])

    ae1m = (We1.reshape(D_EDGE, HEADS, OUT_CH) * a_e1[None]).sum(-1)
    ae2m = (We2.reshape(D_EDGE, HEADS, OUT_CH) * a_e2[None]).sum(-1)
    ae1, ae2, mxae1, mxae2, nmx, nmn = _prep0(edge_attr, ae1m, ae2m, nodes)
    pad_blk = jnp.full((npad, HEADS), -1e30, f32)
    ae1p = jnp.concatenate([ae1, pad_blk])
    ae2p = jnp.concatenate([ae2, pad_blk])

    as1 = (W1.reshape(1, HEADS, OUT_CH) * a_src1[None]).sum(-1)[0]
    ad1 = (W1.reshape(1, HEADS, OUT_CH) * a_dst1[None]).sum(-1)[0]
    nmx_, nmn_ = nmx[0, 0], nmn[0, 0]
    m1b = (jnp.maximum(nmx_ * as1, nmn_ * as1)
           + jnp.maximum(nmx_ * ad1, nmn_ * ad1) + mxae1[0])
    m1 = jnp.maximum(m1b, 0.2 * m1b)
    cons1 = _splat16([as1, ad1, m1])

    z8 = jnp.zeros((ZROWS, 2 * HEADS), f32)
    pq_part = _make_sc1()(srcp, dstp, ae1p, nodes.reshape(-1), cons1, z8)

    zpad = jnp.zeros((ACC_ROWS - NODE_NUM, HID), f32)
    dm1 = jnp.concatenate([jax.random.bernoulli(
        jax.random.key(1), 0.5, (NODE_NUM, HID)).astype(f32) * 2.0, zpad])
    dm2 = jnp.concatenate([jax.random.bernoulli(
        jax.random.key(2), 0.5, (NODE_NUM, HID)).astype(f32) * 2.0, zpad])
    e4 = _expander()
    as2m = _head_embed(a_src2)
    ad2m = _head_embed(a_dst2)
    h2, as2, ad2, mxs2, mxd2 = _prep2(pq_part, W1, b1.reshape(1, -1), dm1,
                                      e4, W2, as2m, ad2m)
    m2b = mxs2[0] + mxd2[0] + mxae2[0]
    m2 = jnp.maximum(m2b, 0.2 * m2b)
    m2v = _splat16([m2])

    z36 = jnp.zeros((ZROWS, 40), f32)
    pm_part = _make_sc2()(srcp, dstp, ae2p, h2, as2.reshape(-1),
                          ad2.reshape(-1), m2v, z36)

    x2 = _prep3(pm_part, b2.reshape(1, -1), dm2, e4)
    xf = x2[:NODE_NUM].reshape(1, -1)
    maskf = action_mask.astype(f32)
    probs, value = _dense_head(xf, Wl1, bl1.reshape(1, -1), Wl2,
                               bl2.reshape(1, -1), Wc1, bc1.reshape(1, -1),
                               Wc2, bc2.reshape(1, 1), maskf)
    action = jax.random.categorical(jax.random.key(3), jnp.log(probs + 1e-12),
                                    axis=-1)
    return (probs, value, action)
